# baseline (speedup 1.0000x reference)
import jax
import jax.numpy as jnp
from jax.experimental import pallas as pl

_B = 4
_TS = 288
_DIM = 768
_K = int(_TS * _TS * 0.25)


def _gnn_body(x_ref, at_ref, w1_ref, b1_ref, w2_ref, b2_ref, wd_ref,
              bd_ref, out_ref):
    AT = at_ref[...]
    nodes = x_ref[:, 1 + _TS:, :].reshape(_B * _TS, _DIM)

    bits = jax.lax.bitcast_convert_type(AT, jnp.int32)
    hi = (bits >> 16).astype(jnp.int16)
    lo = ((bits & 0xFFFF) ^ 0x8000).astype(jnp.int16)

    def count16(m):
        while m.shape[1] > 1:
            n = m.shape[1]
            h = n // 2
            rest = m[:, 2 * h:]
            m = m[:, :h] + m[:, h:2 * h]
            if n % 2:
                m = jnp.concatenate([m[:, :1] + rest, m[:, 1:]], axis=1)
        return jnp.sum(m.astype(jnp.int32), axis=(1, 2)).reshape(_B, 1, 1)

    res_hi = jnp.zeros((_B, 1, 1), jnp.int16)
    one16 = jnp.int16(1)
    zero16 = jnp.int16(0)
    for bit in range(13, -1, -1):
        cand = res_hi | jnp.int16(1 << bit)
        cnt = count16(jnp.where(hi >= cand, one16, zero16))
        res_hi = jnp.where(cnt >= _K, cand, res_hi)

    eqm = jnp.where(hi == res_hi, one16, zero16)
    c_hi = count16(jnp.where(hi > res_hi, one16, zero16))

    res_lo = jnp.full((_B, 1, 1), -0x8000, jnp.int16)
    for bit in range(15, -1, -1):
        if bit == 15:
            cand = jnp.zeros((_B, 1, 1), jnp.int16)
        else:
            cand = res_lo | jnp.int16(1 << bit)
        cnt = c_hi + count16(jnp.where(lo >= cand, eqm, zero16))
        res_lo = jnp.where(cnt >= _K, cand, res_lo)

    res = (res_hi.astype(jnp.int32) << 16) | (
        (res_lo.astype(jnp.int32) ^ 0x8000) & 0xFFFF)

    S_T = jnp.where(bits >= res, AT, 0.0)
    deg = 1.0 + jnp.sum(S_T, axis=2, keepdims=True)
    dinv3 = 1.0 / jnp.sqrt(deg)
    dinv = dinv3.reshape(_B * _TS, 1)
    dinv2 = dinv * dinv

    def gcn(h, w_ref, b_ref):
        xw = jnp.dot(h, w_ref[...], preferred_element_type=jnp.float32)
        y = (dinv * xw).reshape(_B, _TS, _DIM)
        agg = jax.lax.dot_general(
            S_T, y, (((2,), (1,)), ((0,), (0,))),
            preferred_element_type=jnp.float32).reshape(_B * _TS, _DIM)
        return dinv * agg + dinv2 * xw + b_ref[...]

    h1 = jnp.maximum(gcn(nodes, w1_ref, b1_ref), 0.0)
    h2 = jnp.maximum(gcn(h1, w2_ref, b2_ref), 0.0)

    d = jnp.sum(h2 * wd_ref[...], axis=1, keepdims=True) + bd_ref[0, 0]
    p1 = 1.0 / (1.0 + jnp.exp(-d))
    p0 = 1.0 - p1
    nodes3 = nodes.reshape(_B, _TS, _DIM)
    r0 = jnp.sum(p0.reshape(_B, _TS, 1) * nodes3, axis=1, keepdims=True)
    r1 = jnp.sum(p1.reshape(_B, _TS, 1) * nodes3, axis=1, keepdims=True)
    out_ref[:, 0:_TS, :] = x_ref[:, 1:1 + _TS, :]
    out_ref[:, _TS:_TS + 2, :] = jnp.concatenate([r0, r1], axis=1)


@jax.jit
def kernel(x, attn, W1, b1, W2, b2, Wc, bc):
    A_T = jnp.swapaxes(attn[:, 1 + _TS:, 1 + _TS:], 1, 2)
    wd = (Wc[:, 1] - Wc[:, 0]).reshape(1, _DIM)
    bd = (bc[1] - bc[0]).reshape(1, 1)

    return pl.pallas_call(
        _gnn_body,
        out_shape=jax.ShapeDtypeStruct((_B, _TS + 2, _DIM), jnp.float32),
    )(x, A_T, W1, b1.reshape(1, _DIM), W2, b2.reshape(1, _DIM), wd, bd)


# speedup vs baseline: 1.0763x; 1.0763x over previous
import jax
import jax.numpy as jnp
from jax.experimental import pallas as pl

_B = 4
_TS = 288
_DIM = 768
_K = int(_TS * _TS * 0.25)


def _gnn_body(at_ref, nodes_ref, w1_ref, b1_ref, w2_ref, b2_ref, wd_ref,
              bd_ref, out_ref):
    AT = at_ref[...]
    nodes = nodes_ref[...].reshape(_B * _TS, _DIM)
    ones_c = jnp.ones((_B, _TS, 1), jnp.float32)

    def colsum(m):
        return jax.lax.dot_general(
            m, ones_c, (((2,), (1,)), ((0,), (0,))),
            preferred_element_type=jnp.float32)

    bits = jax.lax.bitcast_convert_type(AT, jnp.int32)
    res = jnp.zeros((_B, 1, 1), jnp.int32)
    kf = jnp.float32(_K)
    for bit in range(30, -1, -1):
        cand = res | jnp.int32(1 << bit)
        m = jnp.where(bits >= cand, 1.0, 0.0)
        cnt = jnp.sum(colsum(m), axis=(1, 2), keepdims=True)
        res = jnp.where(cnt >= kf, cand, res)

    S_T = jnp.where(bits >= res, AT, 0.0)
    deg = 1.0 + colsum(S_T)
    dinv3 = 1.0 / jnp.sqrt(deg)
    dinv = dinv3.reshape(_B * _TS, 1)
    dinv2 = dinv * dinv

    def gcn(h, w_ref, b_ref):
        xw = jnp.dot(h, w_ref[...], preferred_element_type=jnp.float32)
        y = (dinv * xw).reshape(_B, _TS, _DIM)
        agg = jax.lax.dot_general(
            S_T, y, (((2,), (1,)), ((0,), (0,))),
            preferred_element_type=jnp.float32).reshape(_B * _TS, _DIM)
        return dinv * agg + dinv2 * xw + b_ref[...]

    h1 = jnp.maximum(gcn(nodes, w1_ref, b1_ref), 0.0)
    h2 = jnp.maximum(gcn(h1, w2_ref, b2_ref), 0.0)

    d = jnp.dot(h2, wd_ref[...], preferred_element_type=jnp.float32)
    p1 = 1.0 / (1.0 + jnp.exp(-(d + bd_ref[0, 0])))
    P = jnp.concatenate([1.0 - p1, p1], axis=1).reshape(_B, _TS, 2)
    out_ref[...] = jax.lax.dot_general(
        P, nodes.reshape(_B, _TS, _DIM), (((1,), (1,)), ((0,), (0,))),
        preferred_element_type=jnp.float32)


@jax.jit
def kernel(x, attn, W1, b1, W2, b2, Wc, bc):
    n = _TS
    non_skip_tk = x[:, 1:1 + n]
    skip_tk = x[:, 1 + n:]
    A_T = jnp.swapaxes(attn[:, 1 + n:, 1 + n:], 1, 2)

    wd = (Wc[:, 1] - Wc[:, 0]).reshape(_DIM, 1)
    bd = (bc[1] - bc[0]).reshape(1, 1)

    summaries = pl.pallas_call(
        _gnn_body,
        out_shape=jax.ShapeDtypeStruct((_B, 2, _DIM), jnp.float32),
    )(A_T, skip_tk, W1, b1.reshape(1, _DIM), W2, b2.reshape(1, _DIM), wd, bd)

    return jnp.concatenate([non_skip_tk, summaries], axis=1)
